# baseline (speedup 1.0000x reference)
import jax
import jax.numpy as jnp
from jax import lax
from jax.experimental import pallas as pl
from jax.experimental.pallas import tpu as pltpu
from jax.experimental.pallas import tpu_sc as plsc

_B = 4096
_L = 200
_DIM = 50
_DPAD = 64
_N = _B * _L

_NC = 2
_NS = 16
_NW = _NC * _NS

_IW = 128
_IROWS = _N // _IW
_IROWS_PER_W = _IROWS // _NW
_TILE = 8
_NCHUNK = _IROWS_PER_W // _TILE
_CHUNK = _TILE * _IW


def _sc_body(idx_hbm, table_hbm, out_hbm, idx_v, rows_v, sem):
    wid = lax.axis_index("s") * _NC + lax.axis_index("c")
    base = wid * _IROWS_PER_W

    def step(i, carry):
        roff = base + i * _TILE
        pltpu.sync_copy(idx_hbm.at[pl.ds(roff, _TILE)], idx_v)
        for j in range(_TILE):
            pltpu.async_copy(
                table_hbm.at[idx_v.at[j]],
                rows_v.at[pl.ds(j * _IW, _IW)],
                sem,
            )
        for j in range(_TILE):
            pltpu.make_async_copy(
                table_hbm.at[idx_v.at[j]],
                rows_v.at[pl.ds(j * _IW, _IW)],
                sem,
            ).wait()
        pltpu.sync_copy(rows_v, out_hbm.at[pl.ds(roff * _IW, _CHUNK)])
        return carry

    lax.fori_loop(0, _NCHUNK, step, 0)


@jax.jit
def _sc_gather(idx2d, table_pad):
    fn = pl.kernel(
        _sc_body,
        mesh=plsc.VectorSubcoreMesh(core_axis_name="c", subcore_axis_name="s"),
        out_type=jax.ShapeDtypeStruct((_N, _DPAD), jnp.float32),
        scratch_types=[
            pltpu.VMEM((_TILE, _IW), jnp.int32),
            pltpu.VMEM((_CHUNK, _DPAD), jnp.float32),
            pltpu.SemaphoreType.DMA,
        ],
        compiler_params=pltpu.CompilerParams(use_tc_tiling_on_sc=False),
    )
    return fn(idx2d, table_pad)


def kernel(indices, table):
    idx2d = indices.reshape(_IROWS, _IW).astype(jnp.int32)
    table_pad = jnp.pad(table, ((0, 0), (0, _DPAD - _DIM)))
    out = _sc_gather(idx2d, table_pad)
    return out[:, :_DIM].reshape(_B, _L, _DIM)
